# baseline (speedup 1.0000x reference)
import functools

import jax
import jax.numpy as jnp
from jax import lax
from jax.experimental import pallas as pl
from jax.experimental.pallas import tpu as pltpu
from jax.experimental.pallas import tpu_sc as plsc

NC = 2
NS = 16
NW = NC * NS

_N = 10000
_E = 320000


def _make_sc_gather(V, D, E, with_counts=False):
    ew = E // NW
    C = 80 if ew % 80 == 0 else 40
    iters = ew // C
    rows_pt = _N // NS
    ZR = 25
    assert ew % C == 0 and C % 8 == 0

    mesh = plsc.VectorSubcoreMesh(core_axis_name="c", subcore_axis_name="s")

    out_type = [jax.ShapeDtypeStruct((E, D), jnp.float32)]
    scratch = [
        pltpu.VMEM((iters, C), jnp.int32),
        pltpu.VMEM((C, D), jnp.float32),
        pltpu.VMEM((C, D), jnp.float32),
        pltpu.SemaphoreType.DMA,
        pltpu.SemaphoreType.DMA,
        pltpu.SemaphoreType.DMA,
        pltpu.SemaphoreType.DMA,
    ]
    if with_counts:
        out_type += [jax.ShapeDtypeStruct((_N, 16), jnp.float32),
                     jax.ShapeDtypeStruct((_N, 16), jnp.float32)]
        scratch += [
            pltpu.VMEM((iters, C), jnp.int32),
            pltpu.VMEM_SHARED((_N, 16), jnp.float32),
            pltpu.VMEM((C, 16), jnp.float32),
            pltpu.SemaphoreType.DMA,
        ]

    def body(*refs):
        if with_counts:
            (table_hbm, idx_hbm, ridx_hbm, out_hbm, cnt_a, cnt_b,
             idx_all, rows0, rows1, g0, g1, w0, w1,
             ridx_all, cacc, ones_v, csem) = refs
        else:
            (table_hbm, idx_hbm, out_hbm,
             idx_all, rows0, rows1, g0, g1, w0, w1) = refs
        cid = lax.axis_index("c")
        tid = lax.axis_index("s")
        wid = tid * NC + cid
        base = wid * ew
        pltpu.sync_copy(idx_hbm.at[wid], idx_all)

        if with_counts:
            pltpu.sync_copy(ridx_hbm.at[wid], ridx_all)
            zero16 = jnp.zeros((16,), jnp.float32)
            one16 = jnp.ones((16,), jnp.float32)
            for r in range(ZR):
                rows0[r, pl.ds(0, 16)] = zero16
            for r in range(C):
                ones_v[r, pl.ds(0, 16)] = one16

            def zinit(j, _):
                r0 = tid * rows_pt + j * ZR
                pltpu.sync_copy(rows0.at[pl.ds(0, ZR), pl.ds(0, 16)],
                                cacc.at[pl.ds(r0, ZR), :])
                return 0

            lax.fori_loop(0, rows_pt // ZR, zinit, 0)
            plsc.subcore_barrier()

        pltpu.async_copy(table_hbm.at[idx_all.at[0]], rows0, g0)

        def phase(j, cur, nxt, gcur, gnxt, wcur, wnxt):
            pltpu.make_async_copy(table_hbm.at[idx_all.at[j]], cur, gcur).wait()
            pltpu.async_copy(cur, out_hbm.at[pl.ds(base + j * C, C), :], wcur)
            if with_counts:
                @pl.when(j >= 1)
                def _():
                    pltpu.make_async_copy(
                        ones_v, cacc.at[ridx_all.at[0]], csem).wait()
                pltpu.async_copy(ones_v, cacc.at[ridx_all.at[j]], csem,
                                 add=True)

            @pl.when(j + 1 < iters)
            def _():
                @pl.when(j >= 1)
                def _():
                    pltpu.make_async_copy(
                        nxt, out_hbm.at[pl.ds(base, C), :], wnxt).wait()
                pltpu.async_copy(table_hbm.at[idx_all.at[j + 1]], nxt, gnxt)

        def loop_body(j, _):
            @pl.when(j % 2 == 0)
            def _():
                phase(j, rows0, rows1, g0, g1, w0, w1)

            @pl.when(j % 2 == 1)
            def _():
                phase(j, rows1, rows0, g1, g0, w1, w0)
            return 0

        lax.fori_loop(0, iters, loop_body, 0)
        pltpu.make_async_copy(rows0, out_hbm.at[pl.ds(base, C), :], w0).wait()
        pltpu.make_async_copy(rows1, out_hbm.at[pl.ds(base, C), :], w1).wait()

        if with_counts:
            pltpu.make_async_copy(ones_v, cacc.at[ridx_all.at[0]], csem).wait()
            plsc.subcore_barrier()

            def drain(j, _):
                r0 = tid * rows_pt + j * ZR
                stg = rows0.at[pl.ds(0, ZR), pl.ds(0, 16)]
                pltpu.sync_copy(cacc.at[pl.ds(r0, ZR), :], stg)

                @pl.when(cid == 0)
                def _():
                    pltpu.sync_copy(stg, cnt_a.at[pl.ds(r0, ZR), :])

                @pl.when(cid == 1)
                def _():
                    pltpu.sync_copy(stg, cnt_b.at[pl.ds(r0, ZR), :])
                return 0

            lax.fori_loop(0, rows_pt // ZR, drain, 0)

    return functools.partial(
        pl.kernel, mesh=mesh, out_type=tuple(out_type) if with_counts
        else out_type[0],
        scratch_types=scratch,
        compiler_params=pltpu.CompilerParams(use_tc_tiling_on_sc=False),
    )(body)


def _make_sc_segsum(E, N, D2):
    ew = E // NS
    C = 80
    iters = ew // C
    rows_pt = N // NS
    ZR = 25
    assert ew % C == 0 and rows_pt % ZR == 0

    mesh = plsc.VectorSubcoreMesh(core_axis_name="c", subcore_axis_name="s")

    out_type = [
        jax.ShapeDtypeStruct((N, D2), jnp.float32),
        jax.ShapeDtypeStruct((N, D2), jnp.float32),
    ]
    scratch = [
        pltpu.VMEM_SHARED((N, D2), jnp.float32),
        pltpu.VMEM((C, D2), jnp.float32),
        pltpu.VMEM((C, D2), jnp.float32),
        pltpu.VMEM((iters, C), jnp.int32),
        pltpu.SemaphoreType.DMA,
        pltpu.SemaphoreType.DMA,
        pltpu.SemaphoreType.DMA,
        pltpu.SemaphoreType.DMA,
    ]

    def body(e_lo, e_hi, recv, out_lo, out_hi,
             acc, eb0, eb1, idx_all, r0s, r1s, s0s, s1s):
        cid = lax.axis_index("c")
        tid = lax.axis_index("s")
        base = tid * ew

        pltpu.sync_copy(recv.at[tid], idx_all)

        def rstart(j, buf, sem):
            @pl.when(cid == 0)
            def _():
                pltpu.async_copy(e_lo.at[pl.ds(base + j * C, C), :], buf, sem)

            @pl.when(cid == 1)
            def _():
                pltpu.async_copy(e_hi.at[pl.ds(base + j * C, C), :], buf, sem)

        def rwait(buf, sem):
            pltpu.make_async_copy(
                e_lo.at[pl.ds(base, C), :], buf, sem).wait()

        zero16 = jnp.zeros((16,), jnp.float32)
        for r in range(ZR):
            for q in range(D2 // 16):
                eb0[r, pl.ds(q * 16, 16)] = zero16

        def zinit(j, _):
            r0 = tid * rows_pt + j * ZR
            pltpu.sync_copy(eb0.at[pl.ds(0, ZR), :], acc.at[pl.ds(r0, ZR), :])
            return 0

        lax.fori_loop(0, rows_pt // ZR, zinit, 0)
        plsc.subcore_barrier()
        rstart(0, eb0, r0s)

        def swait(buf, sem):
            pltpu.make_async_copy(buf, acc.at[idx_all.at[0]], sem).wait()

        def phase(j, cur, nxt, rcur, rnxt, scur, snxt):
            rwait(cur, rcur)
            pltpu.async_copy(cur, acc.at[idx_all.at[j]], scur, add=True)

            @pl.when(j + 1 < iters)
            def _():
                @pl.when(j >= 1)
                def _():
                    swait(nxt, snxt)
                rstart(j + 1, nxt, rnxt)

        def chunk(j, _):
            @pl.when(j % 2 == 0)
            def _():
                phase(j, eb0, eb1, r0s, r1s, s0s, s1s)

            @pl.when(j % 2 == 1)
            def _():
                phase(j, eb1, eb0, r1s, r0s, s1s, s0s)
            return 0

        lax.fori_loop(0, iters, chunk, 0)
        swait(eb0, s0s)
        swait(eb1, s1s)
        plsc.subcore_barrier()

        def drain(j, _):
            r0 = tid * rows_pt + j * ZR
            pltpu.sync_copy(acc.at[pl.ds(r0, ZR), :], eb0.at[pl.ds(0, ZR), :])

            @pl.when(cid == 0)
            def _():
                pltpu.sync_copy(eb0.at[pl.ds(0, ZR), :],
                                out_lo.at[pl.ds(r0, ZR), :])

            @pl.when(cid == 1)
            def _():
                pltpu.sync_copy(eb0.at[pl.ds(0, ZR), :],
                                out_hi.at[pl.ds(r0, ZR), :])
            return 0

        lax.fori_loop(0, rows_pt // ZR, drain, 0)

    return functools.partial(
        pl.kernel, mesh=mesh, out_type=tuple(out_type),
        scratch_types=scratch,
        compiler_params=pltpu.CompilerParams(use_tc_tiling_on_sc=False),
    )(body)


def _make_sc_segsum_esplit(E, N, D):
    ew = E // NW
    C = 80 if ew % 80 == 0 else 40
    iters = ew // C
    rows_pt = N // NS
    ZR = 25
    assert ew % C == 0 and rows_pt % ZR == 0

    mesh = plsc.VectorSubcoreMesh(core_axis_name="c", subcore_axis_name="s")

    out_type = [
        jax.ShapeDtypeStruct((N, D), jnp.float32),
        jax.ShapeDtypeStruct((N, D), jnp.float32),
    ]
    scratch = [
        pltpu.VMEM_SHARED((N, D), jnp.float32),
        pltpu.VMEM((C, D), jnp.float32),
        pltpu.VMEM((C, D), jnp.float32),
        pltpu.VMEM((iters, C), jnp.int32),
        pltpu.SemaphoreType.DMA,
        pltpu.SemaphoreType.DMA,
        pltpu.SemaphoreType.DMA,
        pltpu.SemaphoreType.DMA,
    ]

    def body(e_hbm, recv, out_a, out_b,
             acc, eb0, eb1, idx_all, r0s, r1s, s0s, s1s):
        cid = lax.axis_index("c")
        tid = lax.axis_index("s")
        wid = tid * NC + cid
        base = wid * ew

        pltpu.sync_copy(recv.at[wid], idx_all)

        zero16 = jnp.zeros((16,), jnp.float32)
        for r in range(ZR):
            for q in range(D // 16):
                eb0[r, pl.ds(q * 16, 16)] = zero16

        def zinit(j, _):
            r0 = tid * rows_pt + j * ZR
            pltpu.sync_copy(eb0.at[pl.ds(0, ZR), :], acc.at[pl.ds(r0, ZR), :])
            return 0

        lax.fori_loop(0, rows_pt // ZR, zinit, 0)
        plsc.subcore_barrier()

        def rstart(j, buf, sem):
            pltpu.async_copy(e_hbm.at[pl.ds(base + j * C, C), :], buf, sem)

        def rwait(buf, sem):
            pltpu.make_async_copy(e_hbm.at[pl.ds(base, C), :], buf, sem).wait()

        def swait(buf, sem):
            pltpu.make_async_copy(buf, acc.at[idx_all.at[0]], sem).wait()

        rstart(0, eb0, r0s)

        def phase(j, cur, nxt, rcur, rnxt, scur, snxt):
            rwait(cur, rcur)
            pltpu.async_copy(cur, acc.at[idx_all.at[j]], scur, add=True)

            @pl.when(j + 1 < iters)
            def _():
                @pl.when(j >= 1)
                def _():
                    swait(nxt, snxt)
                rstart(j + 1, nxt, rnxt)

        def chunk(j, _):
            @pl.when(j % 2 == 0)
            def _():
                phase(j, eb0, eb1, r0s, r1s, s0s, s1s)

            @pl.when(j % 2 == 1)
            def _():
                phase(j, eb1, eb0, r1s, r0s, s1s, s0s)
            return 0

        lax.fori_loop(0, iters, chunk, 0)
        swait(eb0, s0s)
        swait(eb1, s1s)
        plsc.subcore_barrier()

        def drain(j, _):
            r0 = tid * rows_pt + j * ZR
            pltpu.sync_copy(acc.at[pl.ds(r0, ZR), :], eb0.at[pl.ds(0, ZR), :])

            @pl.when(cid == 0)
            def _():
                pltpu.sync_copy(eb0.at[pl.ds(0, ZR), :],
                                out_a.at[pl.ds(r0, ZR), :])

            @pl.when(cid == 1)
            def _():
                pltpu.sync_copy(eb0.at[pl.ds(0, ZR), :],
                                out_b.at[pl.ds(r0, ZR), :])
            return 0

        lax.fori_loop(0, rows_pt // ZR, drain, 0)

    return functools.partial(
        pl.kernel, mesh=mesh, out_type=tuple(out_type),
        scratch_types=scratch,
        compiler_params=pltpu.CompilerParams(use_tc_tiling_on_sc=False),
    )(body)


_BE = 8000


def _edge1_body(ea_ref, xs_ref, we_ref, ws_ref, b_ref, lo_ref, hi_ref):
    acc = jnp.dot(ea_ref[...], we_ref[...], preferred_element_type=jnp.float32)
    acc += jnp.dot(xs_ref[...], ws_ref[...], preferred_element_type=jnp.float32)
    e = jnp.maximum(acc + b_ref[...], 0.0)
    lo_ref[...] = e[:, :128]
    hi_ref[...] = e[:, 128:]


def _tc_edge1(edge_attr, off, xs, We1_e, We1_s_pad, be1):
    eh = xs.shape[0]
    grid = eh // _BE
    return pl.pallas_call(
        _edge1_body,
        grid=(grid,),
        in_specs=[
            pl.BlockSpec((_BE, 10), lambda i: (i + off, 0)),
            pl.BlockSpec((_BE, 128), lambda i: (i, 0)),
            pl.BlockSpec((10, 256), lambda i: (0, 0)),
            pl.BlockSpec((128, 256), lambda i: (0, 0)),
            pl.BlockSpec((1, 256), lambda i: (0, 0)),
        ],
        out_specs=[
            pl.BlockSpec((_BE, 128), lambda i: (i, 0)),
            pl.BlockSpec((_BE, 128), lambda i: (i, 0)),
        ],
        out_shape=[
            jax.ShapeDtypeStruct((eh, 128), jnp.float32),
            jax.ShapeDtypeStruct((eh, 128), jnp.float32),
        ],
    )(edge_attr, xs, We1_e, We1_s_pad, be1.reshape(1, 256))


def _edge2_body(lo_ref, hi_ref, g_ref, w_ref, b_ref, out_ref):
    acc = jnp.dot(lo_ref[...], w_ref[:128, :], preferred_element_type=jnp.float32)
    acc += jnp.dot(hi_ref[...], w_ref[128:, :], preferred_element_type=jnp.float32)
    out_ref[...] = jnp.maximum(acc + g_ref[...] + b_ref[...], 0.0)


def _tc_edge2(e1_lo, e1_hi, g2, We2_e, be2):
    eh = g2.shape[0]
    grid = eh // _BE
    return pl.pallas_call(
        _edge2_body,
        grid=(grid,),
        in_specs=[
            pl.BlockSpec((_BE, 128), lambda i: (i, 0)),
            pl.BlockSpec((_BE, 128), lambda i: (i, 0)),
            pl.BlockSpec((_BE, 128), lambda i: (i, 0)),
            pl.BlockSpec((256, 128), lambda i: (0, 0)),
            pl.BlockSpec((1, 128), lambda i: (0, 0)),
        ],
        out_specs=pl.BlockSpec((_BE, 128), lambda i: (i, 0)),
        out_shape=jax.ShapeDtypeStruct((eh, 128), jnp.float32),
    )(e1_lo, e1_hi, g2, We2_e, be2.reshape(1, 128))


def _edge3_body(e2_ref, g_ref, w_ref, b_ref, out_ref, sum_ref):
    i = pl.program_id(0)
    acc = jnp.dot(e2_ref[...], w_ref[...], preferred_element_type=jnp.float32)
    e = jnp.maximum(acc + g_ref[:, :64] + b_ref[...], 0.0)
    out_ref[...] = jnp.concatenate([e, jnp.zeros_like(e)], axis=1)

    @pl.when(i == 0)
    def _():
        sum_ref[...] = jnp.zeros_like(sum_ref)

    sum_ref[...] += jnp.sum(e.reshape(_BE // 8, 8, 64), axis=0)


def _tc_edge3(e2, g3, We3_e, be3):
    eh = g3.shape[0]
    grid = eh // _BE
    return pl.pallas_call(
        _edge3_body,
        grid=(grid,),
        in_specs=[
            pl.BlockSpec((_BE, 128), lambda i: (i, 0)),
            pl.BlockSpec((_BE, 128), lambda i: (i, 0)),
            pl.BlockSpec((128, 64), lambda i: (0, 0)),
            pl.BlockSpec((1, 64), lambda i: (0, 0)),
        ],
        out_specs=[
            pl.BlockSpec((_BE, 128), lambda i: (i, 0)),
            pl.BlockSpec((8, 64), lambda i: (0, 0)),
        ],
        out_shape=[
            jax.ShapeDtypeStruct((eh, 128), jnp.float32),
            jax.ShapeDtypeStruct((8, 64), jnp.float32),
        ],
    )(e2, g3, We3_e, be3.reshape(1, 64))


_BN = 1000


def _node1_body(prev_ref, lo0_ref, lo1_ref, hi0_ref, hi1_ref,
                ca0_ref, cb0_ref, ca1_ref, cb1_ref, wn_ref, wi_ref,
                b_ref, ws_ref, n_ref, m_ref, cnt_ref):
    cnt = jnp.maximum(ca0_ref[:, 0:1] + cb0_ref[:, 0:1]
                      + ca1_ref[:, 0:1] + cb1_ref[:, 0:1], 1.0)
    cnt_ref[...] = cnt + jnp.zeros((1, 16), jnp.float32)
    s = jnp.concatenate([lo0_ref[...] + lo1_ref[...],
                         hi0_ref[...] + hi1_ref[...]], axis=1) / cnt
    acc = jnp.dot(prev_ref[...], wn_ref[...], preferred_element_type=jnp.float32)
    acc += jnp.dot(s, wi_ref[...], preferred_element_type=jnp.float32)
    n = jnp.maximum(acc + b_ref[...], 0.0)
    n_ref[...] = n
    m_ref[...] = jnp.dot(n, ws_ref[...], preferred_element_type=jnp.float32)


def _tc_node1(prev, lo0, lo1, hi0, hi1, c4, Wn, Wi, b, Ws):
    d_prev = prev.shape[1]
    d_out = Wn.shape[1]
    d_m = Ws.shape[1]
    grid = _N // _BN
    blk = lambda d: pl.BlockSpec((_BN, d), lambda i: (i, 0))
    return pl.pallas_call(
        _node1_body,
        grid=(grid,),
        in_specs=[
            blk(d_prev), blk(128), blk(128), blk(128), blk(128),
            blk(16), blk(16), blk(16), blk(16),
            pl.BlockSpec((d_prev, d_out), lambda i: (0, 0)),
            pl.BlockSpec((256, d_out), lambda i: (0, 0)),
            pl.BlockSpec((1, d_out), lambda i: (0, 0)),
            pl.BlockSpec((d_out, d_m), lambda i: (0, 0)),
        ],
        out_specs=[
            pl.BlockSpec((_BN, d_out), lambda i: (i, 0)),
            pl.BlockSpec((_BN, d_m), lambda i: (i, 0)),
            pl.BlockSpec((_BN, 16), lambda i: (i, 0)),
        ],
        out_shape=[
            jax.ShapeDtypeStruct((_N, d_out), jnp.float32),
            jax.ShapeDtypeStruct((_N, d_m), jnp.float32),
            jax.ShapeDtypeStruct((_N, 16), jnp.float32),
        ],
    )(prev, lo0, lo1, hi0, hi1, *c4, Wn, Wi, b.reshape(1, d_out), Ws)


def _node_sum_body(prev_ref, sa_ref, sb_ref, sc_ref, sd_ref, cnt_ref,
                   wn_ref, wi_ref, b_ref, ws_ref, n_ref, m_ref):
    cnt = jnp.maximum(cnt_ref[:, 0:1], 1.0)
    s = (sa_ref[...] + sb_ref[...] + sc_ref[...] + sd_ref[...]) / cnt
    acc = jnp.dot(prev_ref[...], wn_ref[...], preferred_element_type=jnp.float32)
    acc += jnp.dot(s, wi_ref[...], preferred_element_type=jnp.float32)
    n = jnp.maximum(acc + b_ref[...], 0.0)
    n_ref[...] = n
    m_ref[...] = jnp.dot(n, ws_ref[...], preferred_element_type=jnp.float32)


def _tc_node_sum(prev, s4, cnt, Wn, Wi, b, Ws):
    d_prev = prev.shape[1]
    d_s = s4[0].shape[1]
    d_out = Wn.shape[1]
    d_m = Ws.shape[1]
    grid = _N // _BN
    blk = lambda d: pl.BlockSpec((_BN, d), lambda i: (i, 0))
    return pl.pallas_call(
        _node_sum_body,
        grid=(grid,),
        in_specs=[
            blk(d_prev), blk(d_s), blk(d_s), blk(d_s), blk(d_s), blk(16),
            pl.BlockSpec((d_prev, d_out), lambda i: (0, 0)),
            pl.BlockSpec((d_s, d_out), lambda i: (0, 0)),
            pl.BlockSpec((1, d_out), lambda i: (0, 0)),
            pl.BlockSpec((d_out, d_m), lambda i: (0, 0)),
        ],
        out_specs=[
            pl.BlockSpec((_BN, d_out), lambda i: (i, 0)),
            pl.BlockSpec((_BN, d_m), lambda i: (i, 0)),
        ],
        out_shape=[
            jax.ShapeDtypeStruct((_N, d_out), jnp.float32),
            jax.ShapeDtypeStruct((_N, d_m), jnp.float32),
        ],
    )(prev, *s4, cnt, Wn, Wi, b.reshape(1, d_out), Ws)


def _node3_body(prev_ref, sa_ref, sb_ref, sc_ref, sd_ref, cnt_ref,
                esum0_ref, esum1_ref, wn_ref, wi_ref, b_ref,
                wgn_ref, wge_ref, bg_ref, g_ref, acc_ref):
    i = pl.program_id(0)
    cnt = jnp.maximum(cnt_ref[:, 0:1], 1.0)
    s = (sa_ref[...] + sb_ref[...] + sc_ref[...] + sd_ref[...])[:, :64] / cnt
    acc = jnp.dot(prev_ref[...], wn_ref[...], preferred_element_type=jnp.float32)
    acc += jnp.dot(s, wi_ref[...], preferred_element_type=jnp.float32)
    n3 = jnp.maximum(acc + b_ref[...], 0.0)

    @pl.when(i == 0)
    def _():
        acc_ref[...] = jnp.zeros_like(acc_ref)

    acc_ref[...] += jnp.sum(n3.reshape(_BN // 8, 8, 64), axis=0)

    nmean = jnp.sum(acc_ref[...], axis=0, keepdims=True) * (1.0 / _N)
    emean = jnp.sum(esum0_ref[...] + esum1_ref[...], axis=0,
                    keepdims=True) * (1.0 / _E)
    g = jnp.dot(nmean, wgn_ref[...], preferred_element_type=jnp.float32)
    g += jnp.dot(emean, wge_ref[...], preferred_element_type=jnp.float32)
    g_ref[...] = g + bg_ref[...]


def _tc_node3(n2, s4, cnt, e3sums, Wn3_n, Wn3_i, bn3, Wg_n, Wg_e, bg):
    grid = _N // _BN
    blk = lambda d: pl.BlockSpec((_BN, d), lambda i: (i, 0))
    return pl.pallas_call(
        _node3_body,
        grid=(grid,),
        in_specs=[
            blk(128), blk(128), blk(128), blk(128), blk(128), blk(16),
            pl.BlockSpec((8, 64), lambda i: (0, 0)),
            pl.BlockSpec((8, 64), lambda i: (0, 0)),
            pl.BlockSpec((128, 64), lambda i: (0, 0)),
            pl.BlockSpec((64, 64), lambda i: (0, 0)),
            pl.BlockSpec((1, 64), lambda i: (0, 0)),
            pl.BlockSpec((64, 128), lambda i: (0, 0)),
            pl.BlockSpec((64, 128), lambda i: (0, 0)),
            pl.BlockSpec((1, 128), lambda i: (0, 0)),
        ],
        out_specs=pl.BlockSpec((1, 128), lambda i: (0, 0)),
        out_shape=jax.ShapeDtypeStruct((1, 128), jnp.float32),
        scratch_shapes=[pltpu.VMEM((8, 64), jnp.float32)],
    )(n2, *s4, cnt, *e3sums, Wn3_n, Wn3_i, bn3.reshape(1, 64),
      Wg_n, Wg_e, bg.reshape(1, 128))


_EH = _E // 2
_sc_gather128c = _make_sc_gather(_N, 128, _EH, with_counts=True)
_sc_gather128 = _make_sc_gather(_N, 128, _EH)
_sc_segsum128 = _make_sc_segsum(_EH, _N, 128)
_sc_segsum_es = _make_sc_segsum_esplit(_EH, _N, 128)


def kernel(x, edge_attr, edge_index, We1_e, We1_s, be1, Wn1_n, Wn1_i, bn1,
           We2_e, We2_s, be2, Wn2_n, Wn2_i, bn2, We3_e, We3_s, be3,
           Wn3_n, Wn3_i, bn3, Wg_n, Wg_e, bg):
    snd = edge_index[0]
    rcv = edge_index[1]
    snd_nw = [snd[h * _EH:(h + 1) * _EH].reshape(NW, -1, 40) for h in (0, 1)]
    rcv_nw = [rcv[h * _EH:(h + 1) * _EH].reshape(NW, -1, 40) for h in (0, 1)]
    rcv_ns = [rcv[h * _EH:(h + 1) * _EH].reshape(NS, -1, 80) for h in (0, 1)]
    x_pad = jnp.pad(x, ((0, 0), (0, 114)))
    We1_s_pad = jnp.pad(We1_s, ((0, 114), (0, 0)))
    We3_s_pad = jnp.pad(We3_s, ((0, 0), (0, 64)))
    noff = _EH // _BE

    xs0, ca0, cb0 = _sc_gather128c(x_pad, snd_nw[0], rcv_nw[0])
    xs1, ca1, cb1 = _sc_gather128c(x_pad, snd_nw[1], rcv_nw[1])
    e1_lo0, e1_hi0 = _tc_edge1(edge_attr, 0, xs0, We1_e, We1_s_pad, be1)
    e1_lo1, e1_hi1 = _tc_edge1(edge_attr, noff, xs1, We1_e, We1_s_pad, be1)
    s1_lo0, s1_hi0 = _sc_segsum128(e1_lo0, e1_hi0, rcv_ns[0])
    s1_lo1, s1_hi1 = _sc_segsum128(e1_lo1, e1_hi1, rcv_ns[1])
    n1, m1, cnt = _tc_node1(x, s1_lo0, s1_lo1, s1_hi0, s1_hi1,
                            (ca0, cb0, ca1, cb1), Wn1_n, Wn1_i, bn1, We2_s)

    g2_0 = _sc_gather128(m1, snd_nw[0])
    g2_1 = _sc_gather128(m1, snd_nw[1])
    e2_0 = _tc_edge2(e1_lo0, e1_hi0, g2_0, We2_e, be2)
    e2_1 = _tc_edge2(e1_lo1, e1_hi1, g2_1, We2_e, be2)
    s2a0, s2b0 = _sc_segsum_es(e2_0, rcv_nw[0])
    s2a1, s2b1 = _sc_segsum_es(e2_1, rcv_nw[1])
    n2, m2 = _tc_node_sum(n1, (s2a0, s2b0, s2a1, s2b1), cnt,
                          Wn2_n, Wn2_i, bn2, We3_s_pad)

    g3_0 = _sc_gather128(m2, snd_nw[0])
    g3_1 = _sc_gather128(m2, snd_nw[1])
    e3_0, esum0 = _tc_edge3(e2_0, g3_0, We3_e, be3)
    e3_1, esum1 = _tc_edge3(e2_1, g3_1, We3_e, be3)
    s3a0, s3b0 = _sc_segsum_es(e3_0, rcv_nw[0])
    s3a1, s3b1 = _sc_segsum_es(e3_1, rcv_nw[1])
    g = _tc_node3(n2, (s3a0, s3b0, s3a1, s3b1), cnt, (esum0, esum1),
                  Wn3_n, Wn3_i, bn3, Wg_n, Wg_e, bg)
    return g.reshape(128)


# speedup vs baseline: 1.3531x; 1.3531x over previous
import functools

import jax
import jax.numpy as jnp
from jax import lax
from jax.experimental import pallas as pl
from jax.experimental.pallas import tpu as pltpu
from jax.experimental.pallas import tpu_sc as plsc

NC = 2
NS = 16
NW = NC * NS

_N = 10000
_E = 320000


def _make_sc_gather(V, D, E, with_counts=False):
    ew = E // NW
    C = 125
    iters = ew // C
    rows_pt = _N // NS
    ZR = 25
    assert ew % C == 0

    mesh = plsc.VectorSubcoreMesh(core_axis_name="c", subcore_axis_name="s")

    out_type = [jax.ShapeDtypeStruct((E, D), jnp.float32)]
    scratch = [
        pltpu.VMEM((iters, C), jnp.int32),
        pltpu.VMEM((C, D), jnp.float32),
        pltpu.VMEM((C, D), jnp.float32),
        pltpu.SemaphoreType.DMA,
        pltpu.SemaphoreType.DMA,
        pltpu.SemaphoreType.DMA,
        pltpu.SemaphoreType.DMA,
    ]
    if with_counts:
        out_type += [jax.ShapeDtypeStruct((_N, 16), jnp.float32),
                     jax.ShapeDtypeStruct((_N, 16), jnp.float32)]
        scratch += [
            pltpu.VMEM((iters, C), jnp.int32),
            pltpu.VMEM_SHARED((_N, 16), jnp.float32),
            pltpu.VMEM((C, 16), jnp.float32),
            pltpu.SemaphoreType.DMA,
        ]

    def body(*refs):
        if with_counts:
            (table_hbm, idx_hbm, ridx_hbm, out_hbm, cnt_a, cnt_b,
             idx_all, rows0, rows1, g0, g1, w0, w1,
             ridx_all, cacc, ones_v, csem) = refs
        else:
            (table_hbm, idx_hbm, out_hbm,
             idx_all, rows0, rows1, g0, g1, w0, w1) = refs
        cid = lax.axis_index("c")
        tid = lax.axis_index("s")
        wid = tid * NC + cid
        base = wid * ew
        pltpu.sync_copy(idx_hbm.at[wid], idx_all)

        if with_counts:
            pltpu.sync_copy(ridx_hbm.at[wid], ridx_all)
            zero16 = jnp.zeros((16,), jnp.float32)
            one16 = jnp.ones((16,), jnp.float32)
            for r in range(ZR):
                rows0[r, pl.ds(0, 16)] = zero16
            for r in range(C):
                ones_v[r, pl.ds(0, 16)] = one16

            def zinit(j, _):
                r0 = tid * rows_pt + j * ZR
                pltpu.sync_copy(rows0.at[pl.ds(0, ZR), pl.ds(0, 16)],
                                cacc.at[pl.ds(r0, ZR), :])
                return 0

            lax.fori_loop(0, rows_pt // ZR, zinit, 0)
            plsc.subcore_barrier()

        pltpu.async_copy(table_hbm.at[idx_all.at[0]], rows0, g0)

        def phase(j, cur, nxt, gcur, gnxt, wcur, wnxt):
            pltpu.make_async_copy(table_hbm.at[idx_all.at[j]], cur, gcur).wait()
            pltpu.async_copy(cur, out_hbm.at[pl.ds(base + j * C, C), :], wcur)
            if with_counts:
                @pl.when(j >= 1)
                def _():
                    pltpu.make_async_copy(
                        ones_v, cacc.at[ridx_all.at[0]], csem).wait()
                pltpu.async_copy(ones_v, cacc.at[ridx_all.at[j]], csem,
                                 add=True)

            @pl.when(j + 1 < iters)
            def _():
                @pl.when(j >= 1)
                def _():
                    pltpu.make_async_copy(
                        nxt, out_hbm.at[pl.ds(base, C), :], wnxt).wait()
                pltpu.async_copy(table_hbm.at[idx_all.at[j + 1]], nxt, gnxt)

        def loop_body(j, _):
            @pl.when(j % 2 == 0)
            def _():
                phase(j, rows0, rows1, g0, g1, w0, w1)

            @pl.when(j % 2 == 1)
            def _():
                phase(j, rows1, rows0, g1, g0, w1, w0)
            return 0

        lax.fori_loop(0, iters, loop_body, 0)
        pltpu.make_async_copy(rows0, out_hbm.at[pl.ds(base, C), :], w0).wait()
        pltpu.make_async_copy(rows1, out_hbm.at[pl.ds(base, C), :], w1).wait()

        if with_counts:
            pltpu.make_async_copy(ones_v, cacc.at[ridx_all.at[0]], csem).wait()
            plsc.subcore_barrier()

            def drain(j, _):
                r0 = tid * rows_pt + j * ZR
                stg = rows0.at[pl.ds(0, ZR), pl.ds(0, 16)]
                pltpu.sync_copy(cacc.at[pl.ds(r0, ZR), :], stg)

                @pl.when(cid == 0)
                def _():
                    pltpu.sync_copy(stg, cnt_a.at[pl.ds(r0, ZR), :])

                @pl.when(cid == 1)
                def _():
                    pltpu.sync_copy(stg, cnt_b.at[pl.ds(r0, ZR), :])
                return 0

            lax.fori_loop(0, rows_pt // ZR, drain, 0)

    return functools.partial(
        pl.kernel, mesh=mesh, out_type=tuple(out_type) if with_counts
        else out_type[0],
        scratch_types=scratch,
        compiler_params=pltpu.CompilerParams(use_tc_tiling_on_sc=False),
    )(body)


def _make_sc_segsum(E, N, D2):
    ew = E // NS
    C = 125
    iters = ew // C
    rows_pt = N // NS
    ZR = 25
    assert ew % C == 0 and rows_pt % ZR == 0

    mesh = plsc.VectorSubcoreMesh(core_axis_name="c", subcore_axis_name="s")

    out_type = [
        jax.ShapeDtypeStruct((N, D2), jnp.float32),
        jax.ShapeDtypeStruct((N, D2), jnp.float32),
    ]
    scratch = [
        pltpu.VMEM_SHARED((N, D2), jnp.float32),
        pltpu.VMEM((C, D2), jnp.float32),
        pltpu.VMEM((C, D2), jnp.float32),
        pltpu.VMEM((iters, C), jnp.int32),
        pltpu.SemaphoreType.DMA,
        pltpu.SemaphoreType.DMA,
        pltpu.SemaphoreType.DMA,
        pltpu.SemaphoreType.DMA,
    ]

    def body(e_lo, e_hi, recv, out_lo, out_hi,
             acc, eb0, eb1, idx_all, r0s, r1s, s0s, s1s):
        cid = lax.axis_index("c")
        tid = lax.axis_index("s")
        base = tid * ew

        pltpu.sync_copy(recv.at[tid], idx_all)

        def rstart(j, buf, sem):
            @pl.when(cid == 0)
            def _():
                pltpu.async_copy(e_lo.at[pl.ds(base + j * C, C), :], buf, sem)

            @pl.when(cid == 1)
            def _():
                pltpu.async_copy(e_hi.at[pl.ds(base + j * C, C), :], buf, sem)

        def rwait(buf, sem):
            pltpu.make_async_copy(
                e_lo.at[pl.ds(base, C), :], buf, sem).wait()

        zero16 = jnp.zeros((16,), jnp.float32)
        for r in range(ZR):
            for q in range(D2 // 16):
                eb0[r, pl.ds(q * 16, 16)] = zero16

        def zinit(j, _):
            r0 = tid * rows_pt + j * ZR
            pltpu.sync_copy(eb0.at[pl.ds(0, ZR), :], acc.at[pl.ds(r0, ZR), :])
            return 0

        lax.fori_loop(0, rows_pt // ZR, zinit, 0)
        plsc.subcore_barrier()
        rstart(0, eb0, r0s)

        def swait(buf, sem):
            pltpu.make_async_copy(buf, acc.at[idx_all.at[0]], sem).wait()

        def phase(j, cur, nxt, rcur, rnxt, scur, snxt):
            rwait(cur, rcur)
            pltpu.async_copy(cur, acc.at[idx_all.at[j]], scur, add=True)

            @pl.when(j + 1 < iters)
            def _():
                @pl.when(j >= 1)
                def _():
                    swait(nxt, snxt)
                rstart(j + 1, nxt, rnxt)

        def chunk(j, _):
            @pl.when(j % 2 == 0)
            def _():
                phase(j, eb0, eb1, r0s, r1s, s0s, s1s)

            @pl.when(j % 2 == 1)
            def _():
                phase(j, eb1, eb0, r1s, r0s, s1s, s0s)
            return 0

        lax.fori_loop(0, iters, chunk, 0)
        swait(eb0, s0s)
        swait(eb1, s1s)
        plsc.subcore_barrier()

        def drain(j, _):
            r0 = tid * rows_pt + j * ZR
            pltpu.sync_copy(acc.at[pl.ds(r0, ZR), :], eb0.at[pl.ds(0, ZR), :])

            @pl.when(cid == 0)
            def _():
                pltpu.sync_copy(eb0.at[pl.ds(0, ZR), :],
                                out_lo.at[pl.ds(r0, ZR), :])

            @pl.when(cid == 1)
            def _():
                pltpu.sync_copy(eb0.at[pl.ds(0, ZR), :],
                                out_hi.at[pl.ds(r0, ZR), :])
            return 0

        lax.fori_loop(0, rows_pt // ZR, drain, 0)

    return functools.partial(
        pl.kernel, mesh=mesh, out_type=tuple(out_type),
        scratch_types=scratch,
        compiler_params=pltpu.CompilerParams(use_tc_tiling_on_sc=False),
    )(body)


def _make_sc_segsum_esplit(E, N, D):
    ew = E // NW
    C = 125
    iters = ew // C
    rows_pt = N // NS
    ZR = 25
    assert ew % C == 0 and rows_pt % ZR == 0

    mesh = plsc.VectorSubcoreMesh(core_axis_name="c", subcore_axis_name="s")

    out_type = [
        jax.ShapeDtypeStruct((N, D), jnp.float32),
        jax.ShapeDtypeStruct((N, D), jnp.float32),
    ]
    scratch = [
        pltpu.VMEM_SHARED((N, D), jnp.float32),
        pltpu.VMEM((C, D), jnp.float32),
        pltpu.VMEM((C, D), jnp.float32),
        pltpu.VMEM((iters, C), jnp.int32),
        pltpu.SemaphoreType.DMA,
        pltpu.SemaphoreType.DMA,
        pltpu.SemaphoreType.DMA,
        pltpu.SemaphoreType.DMA,
    ]

    def body(e_hbm, recv, out_a, out_b,
             acc, eb0, eb1, idx_all, r0s, r1s, s0s, s1s):
        cid = lax.axis_index("c")
        tid = lax.axis_index("s")
        wid = tid * NC + cid
        base = wid * ew

        pltpu.sync_copy(recv.at[wid], idx_all)

        zero16 = jnp.zeros((16,), jnp.float32)
        for r in range(ZR):
            for q in range(D // 16):
                eb0[r, pl.ds(q * 16, 16)] = zero16

        def zinit(j, _):
            r0 = tid * rows_pt + j * ZR
            pltpu.sync_copy(eb0.at[pl.ds(0, ZR), :], acc.at[pl.ds(r0, ZR), :])
            return 0

        lax.fori_loop(0, rows_pt // ZR, zinit, 0)
        plsc.subcore_barrier()

        def rstart(j, buf, sem):
            pltpu.async_copy(e_hbm.at[pl.ds(base + j * C, C), :], buf, sem)

        def rwait(buf, sem):
            pltpu.make_async_copy(e_hbm.at[pl.ds(base, C), :], buf, sem).wait()

        def swait(buf, sem):
            pltpu.make_async_copy(buf, acc.at[idx_all.at[0]], sem).wait()

        rstart(0, eb0, r0s)

        def phase(j, cur, nxt, rcur, rnxt, scur, snxt):
            rwait(cur, rcur)
            pltpu.async_copy(cur, acc.at[idx_all.at[j]], scur, add=True)

            @pl.when(j + 1 < iters)
            def _():
                @pl.when(j >= 1)
                def _():
                    swait(nxt, snxt)
                rstart(j + 1, nxt, rnxt)

        def chunk(j, _):
            @pl.when(j % 2 == 0)
            def _():
                phase(j, eb0, eb1, r0s, r1s, s0s, s1s)

            @pl.when(j % 2 == 1)
            def _():
                phase(j, eb1, eb0, r1s, r0s, s1s, s0s)
            return 0

        lax.fori_loop(0, iters, chunk, 0)
        swait(eb0, s0s)
        swait(eb1, s1s)
        plsc.subcore_barrier()

        def drain(j, _):
            r0 = tid * rows_pt + j * ZR
            pltpu.sync_copy(acc.at[pl.ds(r0, ZR), :], eb0.at[pl.ds(0, ZR), :])

            @pl.when(cid == 0)
            def _():
                pltpu.sync_copy(eb0.at[pl.ds(0, ZR), :],
                                out_a.at[pl.ds(r0, ZR), :])

            @pl.when(cid == 1)
            def _():
                pltpu.sync_copy(eb0.at[pl.ds(0, ZR), :],
                                out_b.at[pl.ds(r0, ZR), :])
            return 0

        lax.fori_loop(0, rows_pt // ZR, drain, 0)

    return functools.partial(
        pl.kernel, mesh=mesh, out_type=tuple(out_type),
        scratch_types=scratch,
        compiler_params=pltpu.CompilerParams(use_tc_tiling_on_sc=False),
    )(body)


_BE = 8000


def _edge1_body(ea_ref, xs_ref, we_ref, ws_ref, b_ref, lo_ref, hi_ref):
    acc = jnp.dot(ea_ref[...], we_ref[...], preferred_element_type=jnp.float32)
    acc += jnp.dot(xs_ref[...], ws_ref[...], preferred_element_type=jnp.float32)
    e = jnp.maximum(acc + b_ref[...], 0.0)
    lo_ref[...] = e[:, :128]
    hi_ref[...] = e[:, 128:]


def _tc_edge1(edge_attr, off, xs, We1_e, We1_s_pad, be1):
    eh = xs.shape[0]
    grid = eh // _BE
    return pl.pallas_call(
        _edge1_body,
        grid=(grid,),
        in_specs=[
            pl.BlockSpec((_BE, 10), lambda i: (i + off, 0)),
            pl.BlockSpec((_BE, 128), lambda i: (i, 0)),
            pl.BlockSpec((10, 256), lambda i: (0, 0)),
            pl.BlockSpec((128, 256), lambda i: (0, 0)),
            pl.BlockSpec((1, 256), lambda i: (0, 0)),
        ],
        out_specs=[
            pl.BlockSpec((_BE, 128), lambda i: (i, 0)),
            pl.BlockSpec((_BE, 128), lambda i: (i, 0)),
        ],
        out_shape=[
            jax.ShapeDtypeStruct((eh, 128), jnp.float32),
            jax.ShapeDtypeStruct((eh, 128), jnp.float32),
        ],
    )(edge_attr, xs, We1_e, We1_s_pad, be1.reshape(1, 256))


def _edge2_body(lo_ref, hi_ref, g_ref, w_ref, b_ref, out_ref):
    acc = jnp.dot(lo_ref[...], w_ref[:128, :], preferred_element_type=jnp.float32)
    acc += jnp.dot(hi_ref[...], w_ref[128:, :], preferred_element_type=jnp.float32)
    out_ref[...] = jnp.maximum(acc + g_ref[...] + b_ref[...], 0.0)


def _tc_edge2(e1_lo, e1_hi, g2, We2_e, be2):
    eh = g2.shape[0]
    grid = eh // _BE
    return pl.pallas_call(
        _edge2_body,
        grid=(grid,),
        in_specs=[
            pl.BlockSpec((_BE, 128), lambda i: (i, 0)),
            pl.BlockSpec((_BE, 128), lambda i: (i, 0)),
            pl.BlockSpec((_BE, 128), lambda i: (i, 0)),
            pl.BlockSpec((256, 128), lambda i: (0, 0)),
            pl.BlockSpec((1, 128), lambda i: (0, 0)),
        ],
        out_specs=pl.BlockSpec((_BE, 128), lambda i: (i, 0)),
        out_shape=jax.ShapeDtypeStruct((eh, 128), jnp.float32),
    )(e1_lo, e1_hi, g2, We2_e, be2.reshape(1, 128))


def _edge3_body(e2_ref, g_ref, w_ref, b_ref, out_ref, sum_ref):
    i = pl.program_id(0)
    acc = jnp.dot(e2_ref[...], w_ref[...], preferred_element_type=jnp.float32)
    e = jnp.maximum(acc + g_ref[:, :64] + b_ref[...], 0.0)
    out_ref[...] = jnp.concatenate([e, jnp.zeros_like(e)], axis=1)

    @pl.when(i == 0)
    def _():
        sum_ref[...] = jnp.zeros_like(sum_ref)

    sum_ref[...] += jnp.sum(e.reshape(_BE // 8, 8, 64), axis=0)


def _tc_edge3(e2, g3, We3_e, be3):
    eh = g3.shape[0]
    grid = eh // _BE
    return pl.pallas_call(
        _edge3_body,
        grid=(grid,),
        in_specs=[
            pl.BlockSpec((_BE, 128), lambda i: (i, 0)),
            pl.BlockSpec((_BE, 128), lambda i: (i, 0)),
            pl.BlockSpec((128, 64), lambda i: (0, 0)),
            pl.BlockSpec((1, 64), lambda i: (0, 0)),
        ],
        out_specs=[
            pl.BlockSpec((_BE, 128), lambda i: (i, 0)),
            pl.BlockSpec((8, 64), lambda i: (0, 0)),
        ],
        out_shape=[
            jax.ShapeDtypeStruct((eh, 128), jnp.float32),
            jax.ShapeDtypeStruct((8, 64), jnp.float32),
        ],
    )(e2, g3, We3_e, be3.reshape(1, 64))


_BN = 1000


def _node1_body(prev_ref, lo0_ref, lo1_ref, hi0_ref, hi1_ref,
                ca0_ref, cb0_ref, ca1_ref, cb1_ref, wn_ref, wi_ref,
                b_ref, ws_ref, n_ref, m_ref, cnt_ref):
    cnt = jnp.maximum(ca0_ref[:, 0:1] + cb0_ref[:, 0:1]
                      + ca1_ref[:, 0:1] + cb1_ref[:, 0:1], 1.0)
    cnt_ref[...] = cnt + jnp.zeros((1, 16), jnp.float32)
    s = jnp.concatenate([lo0_ref[...] + lo1_ref[...],
                         hi0_ref[...] + hi1_ref[...]], axis=1) / cnt
    acc = jnp.dot(prev_ref[...], wn_ref[...], preferred_element_type=jnp.float32)
    acc += jnp.dot(s, wi_ref[...], preferred_element_type=jnp.float32)
    n = jnp.maximum(acc + b_ref[...], 0.0)
    n_ref[...] = n
    m_ref[...] = jnp.dot(n, ws_ref[...], preferred_element_type=jnp.float32)


def _tc_node1(prev, lo0, lo1, hi0, hi1, c4, Wn, Wi, b, Ws):
    d_prev = prev.shape[1]
    d_out = Wn.shape[1]
    d_m = Ws.shape[1]
    grid = _N // _BN
    blk = lambda d: pl.BlockSpec((_BN, d), lambda i: (i, 0))
    return pl.pallas_call(
        _node1_body,
        grid=(grid,),
        in_specs=[
            blk(d_prev), blk(128), blk(128), blk(128), blk(128),
            blk(16), blk(16), blk(16), blk(16),
            pl.BlockSpec((d_prev, d_out), lambda i: (0, 0)),
            pl.BlockSpec((256, d_out), lambda i: (0, 0)),
            pl.BlockSpec((1, d_out), lambda i: (0, 0)),
            pl.BlockSpec((d_out, d_m), lambda i: (0, 0)),
        ],
        out_specs=[
            pl.BlockSpec((_BN, d_out), lambda i: (i, 0)),
            pl.BlockSpec((_BN, d_m), lambda i: (i, 0)),
            pl.BlockSpec((_BN, 16), lambda i: (i, 0)),
        ],
        out_shape=[
            jax.ShapeDtypeStruct((_N, d_out), jnp.float32),
            jax.ShapeDtypeStruct((_N, d_m), jnp.float32),
            jax.ShapeDtypeStruct((_N, 16), jnp.float32),
        ],
    )(prev, lo0, lo1, hi0, hi1, *c4, Wn, Wi, b.reshape(1, d_out), Ws)


def _node_sum_body(prev_ref, sa_ref, sb_ref, sc_ref, sd_ref, cnt_ref,
                   wn_ref, wi_ref, b_ref, ws_ref, n_ref, m_ref):
    cnt = jnp.maximum(cnt_ref[:, 0:1], 1.0)
    s = (sa_ref[...] + sb_ref[...] + sc_ref[...] + sd_ref[...]) / cnt
    acc = jnp.dot(prev_ref[...], wn_ref[...], preferred_element_type=jnp.float32)
    acc += jnp.dot(s, wi_ref[...], preferred_element_type=jnp.float32)
    n = jnp.maximum(acc + b_ref[...], 0.0)
    n_ref[...] = n
    m_ref[...] = jnp.dot(n, ws_ref[...], preferred_element_type=jnp.float32)


def _tc_node_sum(prev, s4, cnt, Wn, Wi, b, Ws):
    d_prev = prev.shape[1]
    d_s = s4[0].shape[1]
    d_out = Wn.shape[1]
    d_m = Ws.shape[1]
    grid = _N // _BN
    blk = lambda d: pl.BlockSpec((_BN, d), lambda i: (i, 0))
    return pl.pallas_call(
        _node_sum_body,
        grid=(grid,),
        in_specs=[
            blk(d_prev), blk(d_s), blk(d_s), blk(d_s), blk(d_s), blk(16),
            pl.BlockSpec((d_prev, d_out), lambda i: (0, 0)),
            pl.BlockSpec((d_s, d_out), lambda i: (0, 0)),
            pl.BlockSpec((1, d_out), lambda i: (0, 0)),
            pl.BlockSpec((d_out, d_m), lambda i: (0, 0)),
        ],
        out_specs=[
            pl.BlockSpec((_BN, d_out), lambda i: (i, 0)),
            pl.BlockSpec((_BN, d_m), lambda i: (i, 0)),
        ],
        out_shape=[
            jax.ShapeDtypeStruct((_N, d_out), jnp.float32),
            jax.ShapeDtypeStruct((_N, d_m), jnp.float32),
        ],
    )(prev, *s4, cnt, Wn, Wi, b.reshape(1, d_out), Ws)


def _node3_body(prev_ref, sa_ref, sb_ref, sc_ref, sd_ref, cnt_ref,
                esum0_ref, esum1_ref, wn_ref, wi_ref, b_ref,
                wgn_ref, wge_ref, bg_ref, g_ref, acc_ref):
    i = pl.program_id(0)
    cnt = jnp.maximum(cnt_ref[:, 0:1], 1.0)
    s = (sa_ref[...] + sb_ref[...] + sc_ref[...] + sd_ref[...])[:, :64] / cnt
    acc = jnp.dot(prev_ref[...], wn_ref[...], preferred_element_type=jnp.float32)
    acc += jnp.dot(s, wi_ref[...], preferred_element_type=jnp.float32)
    n3 = jnp.maximum(acc + b_ref[...], 0.0)

    @pl.when(i == 0)
    def _():
        acc_ref[...] = jnp.zeros_like(acc_ref)

    acc_ref[...] += jnp.sum(n3.reshape(_BN // 8, 8, 64), axis=0)

    nmean = jnp.sum(acc_ref[...], axis=0, keepdims=True) * (1.0 / _N)
    emean = jnp.sum(esum0_ref[...] + esum1_ref[...], axis=0,
                    keepdims=True) * (1.0 / _E)
    g = jnp.dot(nmean, wgn_ref[...], preferred_element_type=jnp.float32)
    g += jnp.dot(emean, wge_ref[...], preferred_element_type=jnp.float32)
    g_ref[...] = g + bg_ref[...]


def _tc_node3(n2, s4, cnt, e3sums, Wn3_n, Wn3_i, bn3, Wg_n, Wg_e, bg):
    grid = _N // _BN
    blk = lambda d: pl.BlockSpec((_BN, d), lambda i: (i, 0))
    return pl.pallas_call(
        _node3_body,
        grid=(grid,),
        in_specs=[
            blk(128), blk(128), blk(128), blk(128), blk(128), blk(16),
            pl.BlockSpec((8, 64), lambda i: (0, 0)),
            pl.BlockSpec((8, 64), lambda i: (0, 0)),
            pl.BlockSpec((128, 64), lambda i: (0, 0)),
            pl.BlockSpec((64, 64), lambda i: (0, 0)),
            pl.BlockSpec((1, 64), lambda i: (0, 0)),
            pl.BlockSpec((64, 128), lambda i: (0, 0)),
            pl.BlockSpec((64, 128), lambda i: (0, 0)),
            pl.BlockSpec((1, 128), lambda i: (0, 0)),
        ],
        out_specs=pl.BlockSpec((1, 128), lambda i: (0, 0)),
        out_shape=jax.ShapeDtypeStruct((1, 128), jnp.float32),
        scratch_shapes=[pltpu.VMEM((8, 64), jnp.float32)],
    )(n2, *s4, cnt, *e3sums, Wn3_n, Wn3_i, bn3.reshape(1, 64),
      Wg_n, Wg_e, bg.reshape(1, 128))


_EH = _E // 2
_sc_gather128c = _make_sc_gather(_N, 128, _EH, with_counts=True)
_sc_gather128 = _make_sc_gather(_N, 128, _EH)
_sc_segsum128 = _make_sc_segsum(_EH, _N, 128)
_sc_segsum_es = _make_sc_segsum_esplit(_EH, _N, 128)


def kernel(x, edge_attr, edge_index, We1_e, We1_s, be1, Wn1_n, Wn1_i, bn1,
           We2_e, We2_s, be2, Wn2_n, Wn2_i, bn2, We3_e, We3_s, be3,
           Wn3_n, Wn3_i, bn3, Wg_n, Wg_e, bg):
    snd = edge_index[0]
    rcv = edge_index[1]
    snd_nw = [snd[h * _EH:(h + 1) * _EH].reshape(NW, -1, 125) for h in (0, 1)]
    rcv_nw = [rcv[h * _EH:(h + 1) * _EH].reshape(NW, -1, 125) for h in (0, 1)]
    rcv_ns = [rcv[h * _EH:(h + 1) * _EH].reshape(NS, -1, 125) for h in (0, 1)]
    x_pad = jnp.pad(x, ((0, 0), (0, 114)))
    We1_s_pad = jnp.pad(We1_s, ((0, 114), (0, 0)))
    We3_s_pad = jnp.pad(We3_s, ((0, 0), (0, 64)))
    noff = _EH // _BE

    xs0, ca0, cb0 = _sc_gather128c(x_pad, snd_nw[0], rcv_nw[0])
    xs1, ca1, cb1 = _sc_gather128c(x_pad, snd_nw[1], rcv_nw[1])
    e1_lo0, e1_hi0 = _tc_edge1(edge_attr, 0, xs0, We1_e, We1_s_pad, be1)
    e1_lo1, e1_hi1 = _tc_edge1(edge_attr, noff, xs1, We1_e, We1_s_pad, be1)
    s1_lo0, s1_hi0 = _sc_segsum128(e1_lo0, e1_hi0, rcv_ns[0])
    s1_lo1, s1_hi1 = _sc_segsum128(e1_lo1, e1_hi1, rcv_ns[1])
    n1, m1, cnt = _tc_node1(x, s1_lo0, s1_lo1, s1_hi0, s1_hi1,
                            (ca0, cb0, ca1, cb1), Wn1_n, Wn1_i, bn1, We2_s)

    g2_0 = _sc_gather128(m1, snd_nw[0])
    g2_1 = _sc_gather128(m1, snd_nw[1])
    e2_0 = _tc_edge2(e1_lo0, e1_hi0, g2_0, We2_e, be2)
    e2_1 = _tc_edge2(e1_lo1, e1_hi1, g2_1, We2_e, be2)
    s2a0, s2b0 = _sc_segsum_es(e2_0, rcv_nw[0])
    s2a1, s2b1 = _sc_segsum_es(e2_1, rcv_nw[1])
    n2, m2 = _tc_node_sum(n1, (s2a0, s2b0, s2a1, s2b1), cnt,
                          Wn2_n, Wn2_i, bn2, We3_s_pad)

    g3_0 = _sc_gather128(m2, snd_nw[0])
    g3_1 = _sc_gather128(m2, snd_nw[1])
    e3_0, esum0 = _tc_edge3(e2_0, g3_0, We3_e, be3)
    e3_1, esum1 = _tc_edge3(e2_1, g3_1, We3_e, be3)
    s3a0, s3b0 = _sc_segsum_es(e3_0, rcv_nw[0])
    s3a1, s3b1 = _sc_segsum_es(e3_1, rcv_nw[1])
    g = _tc_node3(n2, (s3a0, s3b0, s3a1, s3b1), cnt, (esum0, esum1),
                  Wn3_n, Wn3_i, bn3, Wg_n, Wg_e, bg)
    return g.reshape(128)
